# baseline (speedup 1.0000x reference)
import functools

import jax
import jax.numpy as jnp
from jax import lax
from jax.experimental import pallas as pl
from jax.experimental.pallas import tpu as pltpu
from jax.experimental.pallas import tpu_sc as plsc

HIDDEN = 768
C = 16
W = 512
B = 1024
BM = 128
NB = B // BM
G = NB + C


def _router_body(h_ref, psi_ref, tc_col_ref,
                 p_c_ref, rank_ref, sidx_ref, hist_ref, off_ref):
    logits = lax.dot_general(
        h_ref[...], psi_ref[...], (((1,), (1,)), ((), ())),
        preferred_element_type=jnp.float32)
    m = jnp.max(logits, axis=1, keepdims=True)
    e = jnp.exp(logits - m)
    p_c_ref[...] = e / jnp.sum(e, axis=1, keepdims=True)

    tcc = tc_col_ref[...]
    cidr = lax.broadcasted_iota(jnp.int32, (1, C), 1)
    oneh = jnp.where(tcc == cidr, 1.0, 0.0)

    bic = lax.broadcasted_iota(jnp.int32, (B, 1), 0)
    bir = lax.broadcasted_iota(jnp.int32, (1, B), 1)
    lt = jnp.where(bir < bic, 1.0, 0.0)
    cs = jnp.dot(lt, oneh, preferred_element_type=jnp.float32)

    hist = jnp.sum(oneh, axis=0, keepdims=True)
    cidc = lax.broadcasted_iota(jnp.int32, (C, 1), 0)
    lt16 = jnp.where(cidc < cidr, 1.0, 0.0)
    off = jnp.dot(hist, lt16, preferred_element_type=jnp.float32)

    rank = jnp.sum((cs + off) * oneh, axis=1, keepdims=True)
    rank_i = rank.astype(jnp.int32)
    rank_ref[...] = rank_i
    hist_ref[...] = hist.astype(jnp.int32)
    off_ref[...] = off.astype(jnp.int32)

    sel = rank_i == bir
    sidx_ref[...] = jnp.sum(jnp.where(sel, bic, 0), axis=0, keepdims=True)


def _group_body(wb_ref, wc_ref, vld_ref, off_ref, hist_ref, last_ref,
                x_ref, w_ref, b_ref, mp_ref, mn_ref, o_ref):
    g = pl.program_id(0)
    c = wc_ref[g]
    blk = wb_ref[g]
    start = off_ref[c]
    cnt = hist_ref[c]
    rows = blk * BM + lax.broadcasted_iota(jnp.int32, (BM, 1), 0)
    rmask = (rows >= start) & (rows < start + cnt)

    @pl.when(vld_ref[g] == 1)
    def _():
        acc = jnp.dot(x_ref[...], w_ref[0], preferred_element_type=jnp.float32,
                      precision=lax.Precision.DEFAULT)
        vals = acc + b_ref[0]
        f = jnp.where(vals > 0, vals, vals * mp_ref[0]) * mn_ref[0]
        o_ref[...] = jnp.where(rmask, f, o_ref[...])

    @pl.when(last_ref[g] == 1)
    def _():
        f = o_ref[...]
        m = jnp.max(f, axis=1, keepdims=True)
        e = jnp.exp(f - m)
        o_ref[...] = e / jnp.sum(e, axis=1, keepdims=True)


def _sc_permute_rows(table, idx, ncols, scatter):
    info = plsc.get_sparse_core_info()
    nw = info.num_cores * info.num_subcores
    bpw = B // nw
    mesh = plsc.VectorSubcoreMesh(core_axis_name="c", subcore_axis_name="s")

    @functools.partial(
        pl.kernel, mesh=mesh,
        out_type=jax.ShapeDtypeStruct((B, ncols), jnp.float32),
        scratch_types=[
            pltpu.VMEM((bpw,), jnp.int32),
            pltpu.VMEM((bpw, ncols), jnp.float32),
            pltpu.SemaphoreType.DMA,
        ],
    )
    def k(table_hbm, idx_hbm, out_hbm, idx_v, rows_v, sem):
        wid = lax.axis_index("s") * info.num_cores + lax.axis_index("c")
        base = wid * bpw
        pltpu.sync_copy(idx_hbm.at[pl.ds(base, bpw)], idx_v)
        if scatter:
            pltpu.sync_copy(table_hbm.at[pl.ds(base, bpw)], rows_v)
            pltpu.async_copy(rows_v, out_hbm.at[idx_v], sem).wait()
        else:
            pltpu.async_copy(table_hbm.at[idx_v], rows_v, sem).wait()
            pltpu.sync_copy(rows_v, out_hbm.at[pl.ds(base, bpw)])

    return k(table, idx)


def kernel(h_p, target_cluster, psi_W, phi_W, phi_b, mask_neg, mask_pos):
    tc = target_cluster.astype(jnp.int32)
    tc_col = tc.reshape(B, 1)

    p_c, rank2, sidx2, hist2, off2 = pl.pallas_call(
        _router_body,
        out_shape=[
            jax.ShapeDtypeStruct((B, C), jnp.float32),
            jax.ShapeDtypeStruct((B, 1), jnp.int32),
            jax.ShapeDtypeStruct((1, B), jnp.int32),
            jax.ShapeDtypeStruct((1, C), jnp.int32),
            jax.ShapeDtypeStruct((1, C), jnp.int32),
        ],
    )(h_p, psi_W, tc_col)

    rank = rank2.reshape(B)
    sidx = sidx2.reshape(B)
    hist = hist2.reshape(C)
    off = off2.reshape(C)

    starts = (jnp.arange(NB, dtype=jnp.int32) * BM)[:, None]
    seg_lo = off[None, :]
    seg_hi = (off + hist)[None, :]
    present = (seg_lo < starts + BM) & (seg_hi > starts) & (hist[None, :] > 0)
    flat = present.reshape(-1)
    pos = jnp.cumsum(flat.astype(jnp.int32)) - 1
    total = pos[-1] + 1
    blk_flat = jnp.repeat(jnp.arange(NB, dtype=jnp.int32), C)
    cl_flat = jnp.tile(jnp.arange(C, dtype=jnp.int32), NB)
    tgt = jnp.where(flat, pos, G)
    wb = jnp.full((G,), NB - 1, jnp.int32).at[tgt].set(blk_flat, mode="drop")
    wc0 = jnp.zeros((G,), jnp.int32).at[tgt].set(cl_flat, mode="drop")
    gi = jnp.arange(G, dtype=jnp.int32)
    wc = jnp.where(gi < total, wc0, jnp.take(wc0, total - 1))
    valid = (gi < total).astype(jnp.int32)
    wb_next = jnp.concatenate([wb[1:], jnp.full((1,), -1, jnp.int32)])
    lastf = (valid * ((gi == total - 1) | (wb_next != wb))).astype(jnp.int32)

    h_sorted = _sc_permute_rows(h_p, sidx, HIDDEN, scatter=False)

    spec = lambda bs, im: pl.BlockSpec(bs, im)
    grid_spec = pltpu.PrefetchScalarGridSpec(
        num_scalar_prefetch=6,
        grid=(G,),
        in_specs=[
            spec((BM, HIDDEN), lambda g, wb, wc, v, o, h, lf: (wb[g], 0)),
            spec((1, HIDDEN, W), lambda g, wb, wc, v, o, h, lf: (wc[g], 0, 0)),
            spec((1, 1, W), lambda g, wb, wc, v, o, h, lf: (wc[g], 0, 0)),
            spec((1, 1, W), lambda g, wb, wc, v, o, h, lf: (wc[g], 0, 0)),
            spec((1, 1, W), lambda g, wb, wc, v, o, h, lf: (wc[g], 0, 0)),
        ],
        out_specs=spec((BM, W), lambda g, wb, wc, v, o, h, lf: (wb[g], 0)),
    )
    p_w_sorted = pl.pallas_call(
        _group_body,
        grid_spec=grid_spec,
        out_shape=jax.ShapeDtypeStruct((B, W), jnp.float32),
    )(wb, wc, valid, off, hist, lastf,
      h_sorted, phi_W, phi_b.reshape(C, 1, W),
      mask_pos.reshape(C, 1, W), mask_neg.reshape(C, 1, W))

    p_w = _sc_permute_rows(p_w_sorted, rank, W, scatter=False)

    return (p_c, p_w)


# speedup vs baseline: 1.0421x; 1.0421x over previous
import functools

import jax
import jax.numpy as jnp
from jax import lax
from jax.experimental import pallas as pl
from jax.experimental.pallas import tpu as pltpu
from jax.experimental.pallas import tpu_sc as plsc

HIDDEN = 768
C = 16
W = 512
B = 1024
BM = 128
NB = B // BM
G = NB + C


def _router_body(h_ref, psi_ref, tc_col_ref,
                 p_c_ref, rank_ref, sidx_ref, hist_ref, off_ref):
    logits = lax.dot_general(
        h_ref[...], psi_ref[...], (((1,), (1,)), ((), ())),
        preferred_element_type=jnp.float32)
    m = jnp.max(logits, axis=1, keepdims=True)
    e = jnp.exp(logits - m)
    p_c_ref[...] = e / jnp.sum(e, axis=1, keepdims=True)

    tcc = tc_col_ref[...]
    cidr = lax.broadcasted_iota(jnp.int32, (1, C), 1)
    oneh = jnp.where(tcc == cidr, 1.0, 0.0)

    bic = lax.broadcasted_iota(jnp.int32, (B, 1), 0)
    bir = lax.broadcasted_iota(jnp.int32, (1, B), 1)
    lt = jnp.where(bir < bic, 1.0, 0.0)
    cs = jnp.dot(lt, oneh, preferred_element_type=jnp.float32)

    hist = jnp.sum(oneh, axis=0, keepdims=True)
    cidc = lax.broadcasted_iota(jnp.int32, (C, 1), 0)
    lt16 = jnp.where(cidc < cidr, 1.0, 0.0)
    off = jnp.dot(hist, lt16, preferred_element_type=jnp.float32)

    rank = jnp.sum((cs + off) * oneh, axis=1, keepdims=True)
    rank_i = rank.astype(jnp.int32)
    rank_ref[...] = rank_i
    hist_ref[...] = hist.astype(jnp.int32)
    off_ref[...] = off.astype(jnp.int32)

    sel = rank_i == bir
    sidx_ref[...] = jnp.sum(jnp.where(sel, bic, 0), axis=0, keepdims=True)


def _group_body(wb_ref, wc_ref, vld_ref, off_ref, hist_ref, last_ref,
                x_ref, w1_ref, w2_ref, b_ref, mp_ref, mn_ref, o_ref):
    g = pl.program_id(0)
    c = wc_ref[g]
    blk = wb_ref[g]
    start = off_ref[c]
    cnt = hist_ref[c]
    rows = blk * BM + lax.broadcasted_iota(jnp.int32, (BM, 1), 0)
    rmask = (rows >= start) & (rows < start + cnt)

    @pl.when(vld_ref[g] == 1)
    def _():
        x = x_ref[...]
        for half, wr in ((0, w1_ref), (1, w2_ref)):
            sl = slice(half * (W // 2), (half + 1) * (W // 2))
            acc = jnp.dot(x, wr[0], preferred_element_type=jnp.float32)
            vals = acc + b_ref[0][:, sl]
            f = jnp.where(vals > 0, vals, vals * mp_ref[0][:, sl])
            f = f * mn_ref[0][:, sl]
            o_ref[:, sl] = jnp.where(rmask, f, o_ref[:, sl])

    @pl.when(last_ref[g] == 1)
    def _():
        f = o_ref[...]
        m = jnp.max(f, axis=1, keepdims=True)
        e = jnp.exp(f - m)
        o_ref[...] = e / jnp.sum(e, axis=1, keepdims=True)


def _sc_permute_rows(table, idx, ncols, scatter):
    info = plsc.get_sparse_core_info()
    nw = info.num_cores * info.num_subcores
    bpw = B // nw
    mesh = plsc.VectorSubcoreMesh(core_axis_name="c", subcore_axis_name="s")

    @functools.partial(
        pl.kernel, mesh=mesh,
        out_type=jax.ShapeDtypeStruct((B, ncols), jnp.float32),
        scratch_types=[
            pltpu.VMEM((bpw,), jnp.int32),
            pltpu.VMEM((bpw, ncols), jnp.float32),
            pltpu.SemaphoreType.DMA,
        ],
    )
    def k(table_hbm, idx_hbm, out_hbm, idx_v, rows_v, sem):
        wid = lax.axis_index("s") * info.num_cores + lax.axis_index("c")
        base = wid * bpw
        pltpu.sync_copy(idx_hbm.at[pl.ds(base, bpw)], idx_v)
        if scatter:
            pltpu.sync_copy(table_hbm.at[pl.ds(base, bpw)], rows_v)
            pltpu.async_copy(rows_v, out_hbm.at[idx_v], sem).wait()
        else:
            pltpu.async_copy(table_hbm.at[idx_v], rows_v, sem).wait()
            pltpu.sync_copy(rows_v, out_hbm.at[pl.ds(base, bpw)])

    return k(table, idx)


def kernel(h_p, target_cluster, psi_W, phi_W, phi_b, mask_neg, mask_pos):
    tc = target_cluster.astype(jnp.int32)
    tc_col = tc.reshape(B, 1)

    p_c, rank2, sidx2, hist2, off2 = pl.pallas_call(
        _router_body,
        out_shape=[
            jax.ShapeDtypeStruct((B, C), jnp.float32),
            jax.ShapeDtypeStruct((B, 1), jnp.int32),
            jax.ShapeDtypeStruct((1, B), jnp.int32),
            jax.ShapeDtypeStruct((1, C), jnp.int32),
            jax.ShapeDtypeStruct((1, C), jnp.int32),
        ],
    )(h_p, psi_W, tc_col)

    rank = rank2.reshape(B)
    sidx = sidx2.reshape(B)
    hist = hist2.reshape(C)
    off = off2.reshape(C)

    seg_end = off + hist
    ev = jnp.arange(G, dtype=jnp.int32)
    is_blk = ev < NB
    ce = jnp.clip(ev - NB, 0, C - 1)
    off_e = jnp.take(off, ce)
    hist_e = jnp.take(hist, ce)
    ce_valid = (~is_blk) & (hist_e > 0) & (off_e % BM != 0)
    pos_e = jnp.where(is_blk, ev * BM,
                      jnp.where(ce_valid, off_e, B + BM + ev))
    def cluster_at(p):
        return jnp.sum(jnp.where(seg_end[None, :] <= p[:, None], 1, 0),
                       axis=1).astype(jnp.int32)
    blk_e = jnp.where(is_blk, ev, jnp.minimum(off_e // BM, NB - 1))
    blk_e = jnp.where(ce_valid | is_blk, blk_e, NB - 1)
    c_last = jnp.sum(jnp.where(seg_end <= B - 1, 1, 0)).astype(jnp.int32)
    c_e = jnp.where(is_blk, cluster_at(ev * BM),
                    jnp.where(ce_valid, ce, c_last))
    g_e = jnp.sum(jnp.where(pos_e[None, :] < pos_e[:, None], 1, 0),
                  axis=1).astype(jnp.int32)
    onehot_g = jnp.where(g_e[:, None] == ev[None, :], 1, 0)
    wb = jnp.sum(onehot_g * blk_e[:, None], axis=0).astype(jnp.int32)
    wc = jnp.sum(onehot_g * c_e[:, None], axis=0).astype(jnp.int32)
    total = NB + jnp.sum(ce_valid.astype(jnp.int32))
    valid = (ev < total).astype(jnp.int32)
    wb_next = jnp.concatenate([wb[1:], jnp.full((1,), -1, jnp.int32)])
    lastf = (valid * ((ev == total - 1) | (wb_next != wb))).astype(jnp.int32)

    h_sorted = _sc_permute_rows(h_p, sidx, HIDDEN, scatter=False)

    spec = lambda bs, im: pl.BlockSpec(bs, im)
    grid_spec = pltpu.PrefetchScalarGridSpec(
        num_scalar_prefetch=6,
        grid=(G,),
        in_specs=[
            spec((BM, HIDDEN), lambda g, wb, wc, v, o, h, lf: (wb[g], 0)),
            spec((1, HIDDEN, W // 2),
                 lambda g, wb, wc, v, o, h, lf: (wc[g], 0, 0)),
            spec((1, HIDDEN, W // 2),
                 lambda g, wb, wc, v, o, h, lf: (wc[g], 0, 1)),
            spec((1, 1, W), lambda g, wb, wc, v, o, h, lf: (wc[g], 0, 0)),
            spec((1, 1, W), lambda g, wb, wc, v, o, h, lf: (wc[g], 0, 0)),
            spec((1, 1, W), lambda g, wb, wc, v, o, h, lf: (wc[g], 0, 0)),
        ],
        out_specs=spec((BM, W), lambda g, wb, wc, v, o, h, lf: (wb[g], 0)),
    )
    p_w_sorted = pl.pallas_call(
        _group_body,
        grid_spec=grid_spec,
        out_shape=jax.ShapeDtypeStruct((B, W), jnp.float32),
    )(wb, wc, valid, off, hist, lastf,
      h_sorted, phi_W, phi_W, phi_b.reshape(C, 1, W),
      mask_pos.reshape(C, 1, W), mask_neg.reshape(C, 1, W))

    p_w = _sc_permute_rows(p_w_sorted, rank, W, scatter=False)

    return (p_c, p_w)


# speedup vs baseline: 1.0427x; 1.0006x over previous
import functools

import jax
import jax.numpy as jnp
from jax import lax
from jax.experimental import pallas as pl
from jax.experimental.pallas import tpu as pltpu
from jax.experimental.pallas import tpu_sc as plsc

HIDDEN = 768
C = 16
W = 512
B = 1024
BM = 128
NB = B // BM
G = NB + C


def _router_body(h_ref, psi_ref, tc_col_ref,
                 p_c_ref, rank_ref, sidx_ref, hist_ref, off_ref):
    logits = lax.dot_general(
        h_ref[...], psi_ref[...], (((1,), (1,)), ((), ())),
        preferred_element_type=jnp.float32)
    m = jnp.max(logits, axis=1, keepdims=True)
    e = jnp.exp(logits - m)
    p_c_ref[...] = e / jnp.sum(e, axis=1, keepdims=True)

    tcc = tc_col_ref[...]
    cidr = lax.broadcasted_iota(jnp.int32, (1, C), 1)
    oneh = jnp.where(tcc == cidr, 1.0, 0.0)

    bic = lax.broadcasted_iota(jnp.int32, (B, 1), 0)
    bir = lax.broadcasted_iota(jnp.int32, (1, B), 1)
    lt = jnp.where(bir < bic, 1.0, 0.0)
    cs = jnp.dot(lt, oneh, preferred_element_type=jnp.float32)

    hist = jnp.sum(oneh, axis=0, keepdims=True)
    cidc = lax.broadcasted_iota(jnp.int32, (C, 1), 0)
    lt16 = jnp.where(cidc < cidr, 1.0, 0.0)
    off = jnp.dot(hist, lt16, preferred_element_type=jnp.float32)

    rank = jnp.sum((cs + off) * oneh, axis=1, keepdims=True)
    rank_i = rank.astype(jnp.int32)
    rank_ref[...] = rank_i
    hist_ref[...] = hist.astype(jnp.int32)
    off_ref[...] = off.astype(jnp.int32)

    sel = rank_i == bir
    sidx_ref[...] = jnp.sum(jnp.where(sel, bic, 0), axis=0, keepdims=True)


def _group_body(wb_ref, wc_ref, vld_ref, off_ref, hist_ref, last_ref,
                x_ref, w1_ref, w2_ref, b_ref, mp_ref, mn_ref, o_ref):
    g = pl.program_id(0)
    c = wc_ref[g]
    blk = wb_ref[g]
    start = off_ref[c]
    cnt = hist_ref[c]
    rows = blk * BM + lax.broadcasted_iota(jnp.int32, (BM, 1), 0)
    rmask = (rows >= start) & (rows < start + cnt)

    @pl.when(vld_ref[g] == 1)
    def _():
        x = x_ref[...].astype(jnp.bfloat16)
        for half, wr in ((0, w1_ref), (1, w2_ref)):
            sl = slice(half * (W // 2), (half + 1) * (W // 2))
            acc = jnp.dot(x, wr[0].astype(jnp.bfloat16),
                          preferred_element_type=jnp.float32)
            vals = acc + b_ref[0][:, sl]
            f = jnp.where(vals > 0, vals, vals * mp_ref[0][:, sl])
            f = f * mn_ref[0][:, sl]
            o_ref[:, sl] = jnp.where(rmask, f, o_ref[:, sl])

    @pl.when(last_ref[g] == 1)
    def _():
        f = o_ref[...]
        m = jnp.max(f, axis=1, keepdims=True)
        e = jnp.exp(f - m)
        o_ref[...] = e / jnp.sum(e, axis=1, keepdims=True)


def _sc_permute_rows(table, idx, ncols, scatter):
    info = plsc.get_sparse_core_info()
    nw = info.num_cores * info.num_subcores
    bpw = B // nw
    mesh = plsc.VectorSubcoreMesh(core_axis_name="c", subcore_axis_name="s")

    @functools.partial(
        pl.kernel, mesh=mesh,
        out_type=jax.ShapeDtypeStruct((B, ncols), jnp.float32),
        scratch_types=[
            pltpu.VMEM((bpw,), jnp.int32),
            pltpu.VMEM((bpw, ncols), jnp.float32),
            pltpu.SemaphoreType.DMA,
        ],
    )
    def k(table_hbm, idx_hbm, out_hbm, idx_v, rows_v, sem):
        wid = lax.axis_index("s") * info.num_cores + lax.axis_index("c")
        base = wid * bpw
        pltpu.sync_copy(idx_hbm.at[pl.ds(base, bpw)], idx_v)
        if scatter:
            pltpu.sync_copy(table_hbm.at[pl.ds(base, bpw)], rows_v)
            pltpu.async_copy(rows_v, out_hbm.at[idx_v], sem).wait()
        else:
            pltpu.async_copy(table_hbm.at[idx_v], rows_v, sem).wait()
            pltpu.sync_copy(rows_v, out_hbm.at[pl.ds(base, bpw)])

    return k(table, idx)


def kernel(h_p, target_cluster, psi_W, phi_W, phi_b, mask_neg, mask_pos):
    tc = target_cluster.astype(jnp.int32)
    tc_col = tc.reshape(B, 1)

    p_c, rank2, sidx2, hist2, off2 = pl.pallas_call(
        _router_body,
        out_shape=[
            jax.ShapeDtypeStruct((B, C), jnp.float32),
            jax.ShapeDtypeStruct((B, 1), jnp.int32),
            jax.ShapeDtypeStruct((1, B), jnp.int32),
            jax.ShapeDtypeStruct((1, C), jnp.int32),
            jax.ShapeDtypeStruct((1, C), jnp.int32),
        ],
    )(h_p, psi_W, tc_col)

    rank = rank2.reshape(B)
    sidx = sidx2.reshape(B)
    hist = hist2.reshape(C)
    off = off2.reshape(C)

    seg_end = off + hist
    ev = jnp.arange(G, dtype=jnp.int32)
    is_blk = ev < NB
    ce = jnp.clip(ev - NB, 0, C - 1)
    off_e = jnp.take(off, ce)
    hist_e = jnp.take(hist, ce)
    ce_valid = (~is_blk) & (hist_e > 0) & (off_e % BM != 0)
    pos_e = jnp.where(is_blk, ev * BM,
                      jnp.where(ce_valid, off_e, B + BM + ev))
    def cluster_at(p):
        return jnp.sum(jnp.where(seg_end[None, :] <= p[:, None], 1, 0),
                       axis=1).astype(jnp.int32)
    blk_e = jnp.where(is_blk, ev, jnp.minimum(off_e // BM, NB - 1))
    blk_e = jnp.where(ce_valid | is_blk, blk_e, NB - 1)
    c_last = jnp.sum(jnp.where(seg_end <= B - 1, 1, 0)).astype(jnp.int32)
    c_e = jnp.where(is_blk, cluster_at(ev * BM),
                    jnp.where(ce_valid, ce, c_last))
    g_e = jnp.sum(jnp.where(pos_e[None, :] < pos_e[:, None], 1, 0),
                  axis=1).astype(jnp.int32)
    onehot_g = jnp.where(g_e[:, None] == ev[None, :], 1, 0)
    wb = jnp.sum(onehot_g * blk_e[:, None], axis=0).astype(jnp.int32)
    wc = jnp.sum(onehot_g * c_e[:, None], axis=0).astype(jnp.int32)
    total = NB + jnp.sum(ce_valid.astype(jnp.int32))
    valid = (ev < total).astype(jnp.int32)
    wb_next = jnp.concatenate([wb[1:], jnp.full((1,), -1, jnp.int32)])
    lastf = (valid * ((ev == total - 1) | (wb_next != wb))).astype(jnp.int32)

    h_sorted = _sc_permute_rows(h_p, sidx, HIDDEN, scatter=False)

    spec = lambda bs, im: pl.BlockSpec(bs, im)
    grid_spec = pltpu.PrefetchScalarGridSpec(
        num_scalar_prefetch=6,
        grid=(G,),
        in_specs=[
            spec((BM, HIDDEN), lambda g, wb, wc, v, o, h, lf: (wb[g], 0)),
            spec((1, HIDDEN, W // 2),
                 lambda g, wb, wc, v, o, h, lf: (wc[g], 0, 0)),
            spec((1, HIDDEN, W // 2),
                 lambda g, wb, wc, v, o, h, lf: (wc[g], 0, 1)),
            spec((1, 1, W), lambda g, wb, wc, v, o, h, lf: (wc[g], 0, 0)),
            spec((1, 1, W), lambda g, wb, wc, v, o, h, lf: (wc[g], 0, 0)),
            spec((1, 1, W), lambda g, wb, wc, v, o, h, lf: (wc[g], 0, 0)),
        ],
        out_specs=spec((BM, W), lambda g, wb, wc, v, o, h, lf: (wb[g], 0)),
    )
    p_w_sorted = pl.pallas_call(
        _group_body,
        grid_spec=grid_spec,
        out_shape=jax.ShapeDtypeStruct((B, W), jnp.float32),
    )(wb, wc, valid, off, hist, lastf,
      h_sorted, phi_W, phi_W, phi_b.reshape(C, 1, W),
      mask_pos.reshape(C, 1, W), mask_neg.reshape(C, 1, W))

    p_w = _sc_permute_rows(p_w_sorted, rank, W, scatter=False)

    return (p_c, p_w)


# speedup vs baseline: 1.1117x; 1.0662x over previous
import functools

import jax
import jax.numpy as jnp
from jax import lax
from jax.experimental import pallas as pl
from jax.experimental.pallas import tpu as pltpu
from jax.experimental.pallas import tpu_sc as plsc

HIDDEN = 768
C = 16
W = 512
B = 1024
BM = 128
NB = B // BM
G = NB + C


def _router_body(h_ref, psi_ref, tc_col_ref,
                 p_c_ref, rank_ref, sidx_ref):
    logits = lax.dot_general(
        h_ref[...], psi_ref[...], (((1,), (1,)), ((), ())),
        preferred_element_type=jnp.float32)
    m = jnp.max(logits, axis=1, keepdims=True)
    e = jnp.exp(logits - m)
    p_c_ref[...] = e / jnp.sum(e, axis=1, keepdims=True)

    tcc = tc_col_ref[...]
    cidr = lax.broadcasted_iota(jnp.int32, (1, C), 1)
    oneh = jnp.where(tcc == cidr, 1.0, 0.0)

    bic = lax.broadcasted_iota(jnp.int32, (B, 1), 0)
    bir = lax.broadcasted_iota(jnp.int32, (1, B), 1)
    lt = jnp.where(bir < bic, 1.0, 0.0)
    cs = jnp.dot(lt, oneh, preferred_element_type=jnp.float32)

    hist = jnp.sum(oneh, axis=0, keepdims=True)
    cidc = lax.broadcasted_iota(jnp.int32, (C, 1), 0)
    lt16 = jnp.where(cidc < cidr, 1.0, 0.0)
    off = jnp.dot(hist, lt16, preferred_element_type=jnp.float32)

    rank = jnp.sum((cs + off) * oneh, axis=1, keepdims=True)
    rank_i = rank.astype(jnp.int32)
    rank_ref[...] = rank_i

    sel = rank_i == bir
    sidx_ref[...] = jnp.sum(jnp.where(sel, bic, 0), axis=0, keepdims=True)


def _meta_body(tc_col_ref, tc_row_ref,
               wb_ref, wc_ref, vld_ref, lastf_ref, off_ref, hist_ref):
    tcc = tc_col_ref[...]
    tcr = tc_row_ref[...]
    cidr = lax.broadcasted_iota(jnp.int32, (1, C), 1)
    cidc = lax.broadcasted_iota(jnp.int32, (C, 1), 0)
    oneh = jnp.where(tcc == cidr, 1.0, 0.0)
    onehT = jnp.where(tcr == cidc, 1.0, 0.0)
    hist_row = jnp.sum(oneh, axis=0, keepdims=True)
    hist_col = jnp.sum(onehT, axis=1, keepdims=True)
    lt16 = jnp.where(cidc < cidr, 1.0, 0.0)
    m16 = jnp.where(cidr < cidc, 1.0, 0.0)
    off_row = jnp.dot(hist_row, lt16,
                      preferred_element_type=jnp.float32).astype(jnp.int32)
    off_col = jnp.dot(m16, hist_col,
                      preferred_element_type=jnp.float32).astype(jnp.int32)
    hist_row = hist_row.astype(jnp.int32)
    hist_col = hist_col.astype(jnp.int32)
    seg_end_row = off_row + hist_row

    ev_c = lax.broadcasted_iota(jnp.int32, (G, 1), 0)
    ev_r = lax.broadcasted_iota(jnp.int32, (1, G), 1)
    zc = jnp.zeros((NB, 1), jnp.int32)
    zr = jnp.zeros((1, NB), jnp.int32)
    off_ce_c = jnp.concatenate([zc, off_col], axis=0)
    hist_ce_c = jnp.concatenate([zc, hist_col], axis=0)
    off_ce_r = jnp.concatenate([zr, off_row], axis=1)
    hist_ce_r = jnp.concatenate([zr, hist_row], axis=1)
    isb_c = ev_c < NB
    isb_r = ev_r < NB
    cev_c = (~isb_c) & (hist_ce_c > 0) & (off_ce_c % BM != 0)
    cev_r = (~isb_r) & (hist_ce_r > 0) & (off_ce_r % BM != 0)
    pos_c = jnp.where(isb_c, ev_c * BM,
                      jnp.where(cev_c, off_ce_c, B + BM + ev_c))
    pos_r = jnp.where(isb_r, ev_r * BM,
                      jnp.where(cev_r, off_ce_r, B + BM + ev_r))
    g_col = jnp.sum(jnp.where(pos_r < pos_c, 1, 0), axis=1, keepdims=True)
    blk_c = jnp.where(isb_c, ev_c,
                      jnp.where(cev_c, off_ce_c // BM, NB - 1))
    c_at_p = jnp.sum(jnp.where(seg_end_row <= ev_c * BM, 1, 0),
                     axis=1, keepdims=True)
    c_last = jnp.sum(jnp.where(seg_end_row <= B - 1, 1, 0),
                     axis=1, keepdims=True)
    c_e = jnp.where(isb_c, c_at_p, jnp.where(cev_c, ev_c - NB, c_last))
    onehot_g = jnp.where(g_col == ev_r, 1, 0)
    wb_row = jnp.sum(onehot_g * blk_c, axis=0, keepdims=True)
    wc_row = jnp.sum(onehot_g * c_e, axis=0, keepdims=True)
    total = NB + jnp.sum(jnp.where(cev_c, 1, 0), axis=0, keepdims=True)
    valid = jnp.where(ev_r < total, 1, 0)
    wb_next = jnp.concatenate(
        [wb_row[:, 1:], jnp.full((1, 1), -1, jnp.int32)], axis=1)
    lastf = valid * jnp.where((ev_r == total - 1) | (wb_next != wb_row), 1, 0)
    wb_ref[...] = wb_row
    wc_ref[...] = wc_row
    vld_ref[...] = valid
    lastf_ref[...] = lastf
    off_ref[...] = off_col
    hist_ref[...] = hist_col


def _group_body(wb_ref, wc_ref, vld_ref, off_ref, hist_ref, last_ref,
                x_ref, w1_ref, w2_ref, b_ref, mp_ref, mn_ref, o_ref):
    g = pl.program_id(0)
    c = wc_ref[0, g]
    blk = wb_ref[0, g]
    start = off_ref[c, 0]
    cnt = hist_ref[c, 0]
    rows = blk * BM + lax.broadcasted_iota(jnp.int32, (BM, 1), 0)
    rmask = (rows >= start) & (rows < start + cnt)

    @pl.when(vld_ref[0, g] == 1)
    def _():
        x = x_ref[...].astype(jnp.bfloat16)
        for half, wr in ((0, w1_ref), (1, w2_ref)):
            sl = slice(half * (W // 2), (half + 1) * (W // 2))
            acc = jnp.dot(x, wr[0].astype(jnp.bfloat16),
                          preferred_element_type=jnp.float32)
            vals = acc + b_ref[0][:, sl]
            f = jnp.where(vals > 0, vals, vals * mp_ref[0][:, sl])
            f = f * mn_ref[0][:, sl]
            o_ref[:, sl] = jnp.where(rmask, f, o_ref[:, sl])

    @pl.when(last_ref[0, g] == 1)
    def _():
        f = o_ref[...]
        m = jnp.max(f, axis=1, keepdims=True)
        e = jnp.exp(f - m)
        o_ref[...] = e / jnp.sum(e, axis=1, keepdims=True)


def _sc_permute_rows(table, idx, ncols, scatter):
    info = plsc.get_sparse_core_info()
    nw = info.num_cores * info.num_subcores
    bpw = B // nw
    mesh = plsc.VectorSubcoreMesh(core_axis_name="c", subcore_axis_name="s")

    @functools.partial(
        pl.kernel, mesh=mesh,
        out_type=jax.ShapeDtypeStruct((B, ncols), jnp.float32),
        scratch_types=[
            pltpu.VMEM((bpw,), jnp.int32),
            pltpu.VMEM((bpw, ncols), jnp.float32),
            pltpu.SemaphoreType.DMA,
        ],
    )
    def k(table_hbm, idx_hbm, out_hbm, idx_v, rows_v, sem):
        wid = lax.axis_index("s") * info.num_cores + lax.axis_index("c")
        base = wid * bpw
        pltpu.sync_copy(idx_hbm.at[pl.ds(base, bpw)], idx_v)
        if scatter:
            pltpu.sync_copy(table_hbm.at[pl.ds(base, bpw)], rows_v)
            pltpu.async_copy(rows_v, out_hbm.at[idx_v], sem).wait()
        else:
            pltpu.async_copy(table_hbm.at[idx_v], rows_v, sem).wait()
            pltpu.sync_copy(rows_v, out_hbm.at[pl.ds(base, bpw)])

    return k(table, idx)


def kernel(h_p, target_cluster, psi_W, phi_W, phi_b, mask_neg, mask_pos):
    tc = target_cluster.astype(jnp.int32)
    tc_col = tc.reshape(B, 1)

    p_c, rank2, sidx2 = pl.pallas_call(
        _router_body,
        out_shape=[
            jax.ShapeDtypeStruct((B, C), jnp.float32),
            jax.ShapeDtypeStruct((B, 1), jnp.int32),
            jax.ShapeDtypeStruct((1, B), jnp.int32),
        ],
    )(h_p, psi_W, tc_col)

    rank = rank2.reshape(B)
    sidx = sidx2.reshape(B)

    wb, wc, valid, lastf, off_c, hist_c = pl.pallas_call(
        _meta_body,
        out_shape=[
            jax.ShapeDtypeStruct((1, G), jnp.int32),
            jax.ShapeDtypeStruct((1, G), jnp.int32),
            jax.ShapeDtypeStruct((1, G), jnp.int32),
            jax.ShapeDtypeStruct((1, G), jnp.int32),
            jax.ShapeDtypeStruct((C, 1), jnp.int32),
            jax.ShapeDtypeStruct((C, 1), jnp.int32),
        ],
    )(tc_col, tc.reshape(1, B))

    h_sorted = _sc_permute_rows(h_p, sidx, HIDDEN, scatter=False)

    spec = lambda bs, im: pl.BlockSpec(bs, im)
    grid_spec = pltpu.PrefetchScalarGridSpec(
        num_scalar_prefetch=6,
        grid=(G,),
        in_specs=[
            spec((BM, HIDDEN), lambda g, wb, wc, v, o, h, lf: (wb[0, g], 0)),
            spec((1, HIDDEN, W // 2),
                 lambda g, wb, wc, v, o, h, lf: (wc[0, g], 0, 0)),
            spec((1, HIDDEN, W // 2),
                 lambda g, wb, wc, v, o, h, lf: (wc[0, g], 0, 1)),
            spec((1, 1, W), lambda g, wb, wc, v, o, h, lf: (wc[0, g], 0, 0)),
            spec((1, 1, W), lambda g, wb, wc, v, o, h, lf: (wc[0, g], 0, 0)),
            spec((1, 1, W), lambda g, wb, wc, v, o, h, lf: (wc[0, g], 0, 0)),
        ],
        out_specs=spec((BM, W), lambda g, wb, wc, v, o, h, lf: (wb[0, g], 0)),
    )
    p_w_sorted = pl.pallas_call(
        _group_body,
        grid_spec=grid_spec,
        out_shape=jax.ShapeDtypeStruct((B, W), jnp.float32),
    )(wb, wc, valid, off_c, hist_c, lastf,
      h_sorted, phi_W, phi_W, phi_b.reshape(C, 1, W),
      mask_pos.reshape(C, 1, W), mask_neg.reshape(C, 1, W))

    p_w = _sc_permute_rows(p_w_sorted, rank, W, scatter=False)

    return (p_c, p_w)
